# baseline (speedup 1.0000x reference)
import functools

import jax
import jax.numpy as jnp
from jax import lax
from jax.experimental import pallas as pl
from jax.experimental.pallas import tpu as pltpu
from jax.experimental.pallas import tpu_sc as plsc

_WORD_LEN = 5
_EMBED_DIM = 128
_LANES = 16
_NCHUNK = _EMBED_DIM // _LANES

_mesh = plsc.VectorSubcoreMesh(core_axis_name="c", subcore_axis_name="s",
                               num_cores=1, num_subcores=_NCHUNK)


@functools.partial(
    pl.kernel,
    out_type=jax.ShapeDtypeStruct((_EMBED_DIM,), jnp.float32),
    mesh=_mesh,
    scratch_types=[
        pltpu.VMEM((_WORD_LEN,), jnp.int32),
        pltpu.VMEM((_WORD_LEN, _EMBED_DIM), jnp.float32),
        pltpu.VMEM((_WORD_LEN, _EMBED_DIM), jnp.float32),
        pltpu.VMEM((_LANES,), jnp.float32),
        pltpu.SemaphoreType.DMA,
    ],
)
def _word_encoder(word_hbm, table_hbm, pos_hbm, out_hbm,
                  idx_v, rows_v, pos_v, acc_v, sem):
    t = lax.axis_index("s")
    col = pl.multiple_of(t * _LANES, _LANES)

    pos_cp = pltpu.async_copy(pos_hbm, pos_v, sem)
    pltpu.sync_copy(word_hbm, idx_v)
    gather = pltpu.async_copy(table_hbm.at[idx_v], rows_v, sem)
    gather.wait()
    pos_cp.wait()

    s = pl.ds(col, _LANES)
    acc = rows_v[0, s] + pos_v[0, s]
    for j in range(1, _WORD_LEN):
        acc = acc + rows_v[j, s] + pos_v[j, s]
    acc_v[...] = acc
    pltpu.sync_copy(acc_v, out_hbm.at[s])


def kernel(word, letter_table, pos_table):
    return _word_encoder(word, letter_table, pos_table)


# speedup vs baseline: 1.0270x; 1.0270x over previous
import functools

import jax
import jax.numpy as jnp
from jax import lax
from jax.experimental import pallas as pl
from jax.experimental.pallas import tpu as pltpu
from jax.experimental.pallas import tpu_sc as plsc

_WORD_LEN = 5
_EMBED_DIM = 128
_LANES = 16

_mesh = plsc.VectorSubcoreMesh(core_axis_name="c", subcore_axis_name="s",
                               num_cores=1, num_subcores=1)


@functools.partial(
    pl.kernel,
    out_type=jax.ShapeDtypeStruct((_EMBED_DIM,), jnp.float32),
    mesh=_mesh,
    scratch_types=[
        pltpu.VMEM((_WORD_LEN,), jnp.int32),
        pltpu.VMEM((_WORD_LEN, _EMBED_DIM), jnp.float32),
        pltpu.VMEM((_WORD_LEN, _EMBED_DIM), jnp.float32),
        pltpu.VMEM((_EMBED_DIM,), jnp.float32),
        pltpu.SemaphoreType.DMA,
    ],
)
def _word_encoder(word_hbm, table_hbm, pos_hbm, out_hbm,
                  idx_v, rows_v, pos_v, acc_v, sem):
    pos_cp = pltpu.async_copy(pos_hbm, pos_v, sem)
    pltpu.sync_copy(word_hbm, idx_v)
    gather = pltpu.async_copy(table_hbm.at[idx_v], rows_v, sem)
    gather.wait()
    pos_cp.wait()

    def chunk(c, carry):
        s = pl.ds(c * _LANES, _LANES)
        acc = rows_v[0, s] + pos_v[0, s]
        for j in range(1, _WORD_LEN):
            acc = acc + rows_v[j, s] + pos_v[j, s]
        acc_v[s] = acc
        return carry

    lax.fori_loop(0, _EMBED_DIM // _LANES, chunk, 0)
    pltpu.sync_copy(acc_v, out_hbm)


def kernel(word, letter_table, pos_table):
    return _word_encoder(word, letter_table, pos_table)
